# baseline (speedup 1.0000x reference)
import functools

import jax
import jax.numpy as jnp
from jax import lax
from jax.experimental import pallas as pl
from jax.experimental.pallas import tpu as pltpu
from jax.experimental.pallas import tpu_sc as plsc

VOCAB = 1000000
EMB_DIM = 64
NUM_CLASS = 38
BATCH = 16384

PBLK = 2048
N_PBLK = 245
HALF = N_PBLK * PBLK
N_VBLK = (VOCAB + PBLK - 1) // PBLK

NUM_CORES = 2
NUM_SUBCORES = 16
NUM_WORKERS = NUM_CORES * NUM_SUBCORES
B_PER_W = BATCH // NUM_WORKERS
CHUNK = 128
N_CHUNKS = B_PER_W // CHUNK


def _repack_body(a_ref, b_ref, i_ref, y_ref):
    ident = i_ref[...]
    dims = (((0,), (0,)), ((), ()))
    y_ref[:, 0:EMB_DIM] = lax.dot_general(
        a_ref[...], ident, dims, preferred_element_type=jnp.float32
    )
    y_ref[:, EMB_DIM:] = lax.dot_general(
        b_ref[...], ident, dims, preferred_element_type=jnp.float32
    )


@jax.jit
def _tc_repack(emb_t, ident):
    return pl.pallas_call(
        _repack_body,
        grid=(N_PBLK,),
        in_specs=[
            pl.BlockSpec((EMB_DIM, PBLK), lambda i: (0, i)),
            pl.BlockSpec((EMB_DIM, PBLK), lambda i: (0, (i + N_PBLK) % N_VBLK)),
            pl.BlockSpec((EMB_DIM, EMB_DIM), lambda i: (0, 0)),
        ],
        out_specs=pl.BlockSpec((PBLK, 2 * EMB_DIM), lambda i: (i, 0)),
        out_shape=jax.ShapeDtypeStruct((HALF, 2 * EMB_DIM), jnp.float32),
    )(emb_t, emb_t, ident)


def _gather_body(y_hbm, idx_hbm, out_hbm, idx_v, rows_v, sem):
    wid = lax.axis_index("s") * NUM_CORES + lax.axis_index("c")
    pltpu.sync_copy(idx_hbm.at[wid], idx_v)
    copies = []
    for j in range(N_CHUNKS):
        c = pltpu.make_async_copy(y_hbm.at[idx_v.at[j]], rows_v.at[j], sem)
        c.start()
        copies.append(c)
    for c in copies:
        c.wait()
    pltpu.sync_copy(rows_v, out_hbm.at[wid])


@jax.jit
def _sc_gather(y, idx3d):
    mesh = plsc.VectorSubcoreMesh(core_axis_name="c", subcore_axis_name="s")
    return pl.kernel(
        _gather_body,
        out_type=jax.ShapeDtypeStruct(
            (NUM_WORKERS, N_CHUNKS, CHUNK, 2 * EMB_DIM), jnp.float32
        ),
        mesh=mesh,
        scratch_types=[
            pltpu.VMEM((N_CHUNKS, CHUNK), jnp.int32),
            pltpu.VMEM((N_CHUNKS, CHUNK, 2 * EMB_DIM), jnp.float32),
            pltpu.SemaphoreType.DMA,
        ],
    )(y, idx3d)


def _select_linear_body(g_ref, h_ref, w_ref, b_ref, o_ref):
    h = h_ref[...]
    x = jnp.where(h == 0, g_ref[:, 0:EMB_DIM], g_ref[:, EMB_DIM:])
    o_ref[...] = (
        jnp.dot(x, w_ref[...], preferred_element_type=jnp.float32) + b_ref[...]
    )


@jax.jit
def _tc_select_linear(g, h2d, w_t, b2d):
    block = 2048
    grid = (BATCH // block,)
    return pl.pallas_call(
        _select_linear_body,
        grid=grid,
        in_specs=[
            pl.BlockSpec((block, 2 * EMB_DIM), lambda i: (i, 0)),
            pl.BlockSpec((block, 1), lambda i: (i, 0)),
            pl.BlockSpec((EMB_DIM, NUM_CLASS), lambda i: (0, 0)),
            pl.BlockSpec((1, NUM_CLASS), lambda i: (0, 0)),
        ],
        out_specs=pl.BlockSpec((block, NUM_CLASS), lambda i: (i, 0)),
        out_shape=jax.ShapeDtypeStruct((BATCH, NUM_CLASS), jnp.float32),
    )(g, h2d, w_t, b2d)


def kernel(node, emb, fc_w, fc_b):
    node = node.astype(jnp.int32)
    h = (node >= HALF).astype(jnp.int32)
    p = node - h * HALF
    idx3d = p.reshape(NUM_WORKERS, N_CHUNKS, CHUNK)
    h2d = h.reshape(BATCH, 1)
    ident = jnp.eye(EMB_DIM, dtype=jnp.float32)
    y = _tc_repack(emb.T, ident)
    g = _sc_gather(y, idx3d).reshape(BATCH, 2 * EMB_DIM)
    return _tc_select_linear(g, h2d, fc_w.T, fc_b.reshape(1, NUM_CLASS))


# speedup vs baseline: 1.3912x; 1.3912x over previous
import functools

import jax
import jax.numpy as jnp
from jax import lax
from jax.experimental import pallas as pl
from jax.experimental.pallas import tpu as pltpu
from jax.experimental.pallas import tpu_sc as plsc

VOCAB = 1000000
EMB_DIM = 64
NUM_CLASS = 38
BATCH = 16384

PBLK = 2048
N_QBLK = 123
Q = N_QBLK * PBLK
N_VBLK = (VOCAB + PBLK - 1) // PBLK

NUM_CORES = 2
NUM_SUBCORES = 16
NUM_WORKERS = NUM_CORES * NUM_SUBCORES
B_PER_W = BATCH // NUM_WORKERS
CHUNK = 128
N_CHUNKS = B_PER_W // CHUNK


def _pack_pair(lo_f32, hi_f32):
    lo = lax.bitcast_convert_type(lo_f32.astype(jnp.bfloat16), jnp.uint16)
    hi = lax.bitcast_convert_type(hi_f32.astype(jnp.bfloat16), jnp.uint16)
    return lo.astype(jnp.uint32) | lax.shift_left(
        hi.astype(jnp.uint32), jnp.uint32(16)
    )


def _project_body(a_ref, b_ref, c_ref, d_ref, w_ref, bias_ref, y_ref):
    w = w_ref[...]
    bias = bias_ref[...]
    dims = (((0,), (1,)), ((), ()))

    def proj(ref):
        return (
            lax.dot_general(
                ref[...], w, dims, preferred_element_type=jnp.float32
            )
            + bias
        )

    y_ref[:, 0:NUM_CLASS] = _pack_pair(proj(a_ref), proj(b_ref))
    y_ref[:, EMB_DIM : EMB_DIM + NUM_CLASS] = _pack_pair(
        proj(c_ref), proj(d_ref)
    )


@jax.jit
def _tc_project(emb_t, fc_w, b2d):
    return pl.pallas_call(
        _project_body,
        grid=(N_QBLK,),
        in_specs=[
            pl.BlockSpec((EMB_DIM, PBLK), lambda i: (0, i)),
            pl.BlockSpec((EMB_DIM, PBLK), lambda i: (0, i + N_QBLK)),
            pl.BlockSpec((EMB_DIM, PBLK), lambda i: (0, i + 2 * N_QBLK)),
            pl.BlockSpec((EMB_DIM, PBLK), lambda i: (0, (i + 3 * N_QBLK) % N_VBLK)),
            pl.BlockSpec((NUM_CLASS, EMB_DIM), lambda i: (0, 0)),
            pl.BlockSpec((1, NUM_CLASS), lambda i: (0, 0)),
        ],
        out_specs=pl.BlockSpec((PBLK, 2 * EMB_DIM), lambda i: (i, 0)),
        out_shape=jax.ShapeDtypeStruct((Q, 2 * EMB_DIM), jnp.uint32),
    )(emb_t, emb_t, emb_t, emb_t, fc_w, b2d)


def _gather_body(y_hbm, idx_hbm, out_hbm, idx_v, rows_v, sem):
    wid = lax.axis_index("s") * NUM_CORES + lax.axis_index("c")
    pltpu.sync_copy(idx_hbm.at[wid], idx_v)
    copies = []
    for j in range(N_CHUNKS):
        c = pltpu.make_async_copy(y_hbm.at[idx_v.at[j]], rows_v.at[j], sem)
        c.start()
        copies.append(c)
    for c in copies:
        c.wait()
    pltpu.sync_copy(rows_v, out_hbm.at[wid])


@jax.jit
def _sc_gather(y, tidx3d):
    mesh = plsc.VectorSubcoreMesh(core_axis_name="c", subcore_axis_name="s")
    return pl.kernel(
        _gather_body,
        out_type=jax.ShapeDtypeStruct(
            (NUM_WORKERS, N_CHUNKS, CHUNK, 2 * EMB_DIM), jnp.uint32
        ),
        mesh=mesh,
        scratch_types=[
            pltpu.VMEM((N_CHUNKS, CHUNK), jnp.int32),
            pltpu.VMEM((N_CHUNKS, CHUNK, 2 * EMB_DIM), jnp.uint32),
            pltpu.SemaphoreType.DMA,
        ],
    )(y, tidx3d)


def _select_body(g_ref, s_ref, o_ref):
    s = s_ref[...]
    a_w = g_ref[:, 0:NUM_CLASS]
    b_w = g_ref[:, EMB_DIM : EMB_DIM + NUM_CLASS]
    w = jnp.where(s < 2, a_w, b_w)
    lo = lax.rem(w, jnp.uint32(65536))
    hi = lax.shift_right_logical(w, jnp.uint32(16))
    odd = lax.rem(s, 2)
    pick = jnp.where(odd == 0, lo, hi)
    b16 = lax.bitcast_convert_type(
        pick.astype(jnp.uint16), jnp.bfloat16
    )
    o_ref[...] = b16.astype(jnp.float32)


@jax.jit
def _tc_select(g, s2d):
    block = 2048
    grid = (BATCH // block,)
    return pl.pallas_call(
        _select_body,
        grid=grid,
        in_specs=[
            pl.BlockSpec((block, 2 * EMB_DIM), lambda i: (i, 0)),
            pl.BlockSpec((block, 1), lambda i: (i, 0)),
        ],
        out_specs=pl.BlockSpec((block, NUM_CLASS), lambda i: (i, 0)),
        out_shape=jax.ShapeDtypeStruct((BATCH, NUM_CLASS), jnp.float32),
    )(g, s2d)


def kernel(node, emb, fc_w, fc_b):
    node = node.astype(jnp.int32)
    s = node // Q
    t = node - s * Q
    tidx3d = t.reshape(NUM_WORKERS, N_CHUNKS, CHUNK)
    s2d = s.reshape(BATCH, 1)
    y = _tc_project(emb.T, fc_w, fc_b.reshape(1, NUM_CLASS))
    g = _sc_gather(y, tidx3d).reshape(BATCH, 2 * EMB_DIM)
    return _tc_select(g, s2d)


# speedup vs baseline: 1.5889x; 1.1420x over previous
import functools

import jax
import jax.numpy as jnp
from jax import lax
from jax.experimental import pallas as pl
from jax.experimental.pallas import tpu as pltpu
from jax.experimental.pallas import tpu_sc as plsc

VOCAB = 1000000
EMB_DIM = 64
NUM_CLASS = 38
BATCH = 16384

PBLK = 4096
N_QBLK = 62
Q = N_QBLK * PBLK
N_VBLK = (VOCAB + PBLK - 1) // PBLK

NUM_CORES = 2
NUM_SUBCORES = 16
NUM_WORKERS = NUM_CORES * NUM_SUBCORES
B_PER_W = BATCH // NUM_WORKERS
CHUNK = 128
N_CHUNKS = B_PER_W // CHUNK


def _pack_pair(lo_f32, hi_f32):
    lo = lax.bitcast_convert_type(lo_f32.astype(jnp.bfloat16), jnp.uint16)
    hi = lax.bitcast_convert_type(hi_f32.astype(jnp.bfloat16), jnp.uint16)
    return lo.astype(jnp.uint32) | lax.shift_left(
        hi.astype(jnp.uint32), jnp.uint32(16)
    )


def _project_body(a_ref, b_ref, c_ref, d_ref, w_ref, y_ref):
    w = w_ref[...]
    dims = (((0,), (1,)), ((), ()))

    def proj(ref):
        return lax.dot_general(
            ref[...], w, dims, preferred_element_type=jnp.float32
        )

    y_ref[:, 0:NUM_CLASS] = _pack_pair(proj(a_ref), proj(b_ref))
    y_ref[:, EMB_DIM : EMB_DIM + NUM_CLASS] = _pack_pair(
        proj(c_ref), proj(d_ref)
    )


@jax.jit
def _tc_project(emb_t, fc_w):
    return pl.pallas_call(
        _project_body,
        grid=(N_QBLK,),
        in_specs=[
            pl.BlockSpec((EMB_DIM, PBLK), lambda i: (0, i)),
            pl.BlockSpec((EMB_DIM, PBLK), lambda i: (0, i + N_QBLK)),
            pl.BlockSpec((EMB_DIM, PBLK), lambda i: (0, i + 2 * N_QBLK)),
            pl.BlockSpec((EMB_DIM, PBLK), lambda i: (0, (i + 3 * N_QBLK) % N_VBLK)),
            pl.BlockSpec((NUM_CLASS, EMB_DIM), lambda i: (0, 0)),
        ],
        out_specs=pl.BlockSpec((PBLK, 2 * EMB_DIM), lambda i: (i, 0)),
        out_shape=jax.ShapeDtypeStruct((Q, 2 * EMB_DIM), jnp.uint32),
        compiler_params=pltpu.CompilerParams(fuse_transposed_lhs_in_matmul=True),
    )(emb_t, emb_t, emb_t, emb_t, fc_w)


def _gather_body(y_hbm, idx_hbm, out_hbm, idx_v, rows_v, sem):
    wid = lax.axis_index("s") * NUM_CORES + lax.axis_index("c")
    pltpu.sync_copy(idx_hbm.at[wid], idx_v)
    copies = []
    for j in range(N_CHUNKS):
        c = pltpu.make_async_copy(y_hbm.at[idx_v.at[j]], rows_v.at[j], sem)
        c.start()
        copies.append(c)
    for c in copies:
        c.wait()
    pltpu.sync_copy(rows_v, out_hbm.at[wid])


@jax.jit
def _sc_gather(y, tidx3d):
    mesh = plsc.VectorSubcoreMesh(core_axis_name="c", subcore_axis_name="s")
    return pl.kernel(
        _gather_body,
        out_type=jax.ShapeDtypeStruct(
            (NUM_WORKERS, N_CHUNKS, CHUNK, 2 * EMB_DIM), jnp.uint32
        ),
        mesh=mesh,
        scratch_types=[
            pltpu.VMEM((N_CHUNKS, CHUNK), jnp.int32),
            pltpu.VMEM((N_CHUNKS, CHUNK, 2 * EMB_DIM), jnp.uint32),
            pltpu.SemaphoreType.DMA,
        ],
    )(y, tidx3d)


def _select_body(g_ref, s_ref, bias_ref, o_ref):
    s = s_ref[...]
    a_w = g_ref[:, 0:NUM_CLASS]
    b_w = g_ref[:, EMB_DIM : EMB_DIM + NUM_CLASS]
    w = jnp.where(s < 2, a_w, b_w)
    lo = lax.rem(w, jnp.uint32(65536))
    hi = lax.shift_right_logical(w, jnp.uint32(16))
    odd = lax.rem(s, 2)
    pick = jnp.where(odd == 0, lo, hi)
    b16 = lax.bitcast_convert_type(
        pick.astype(jnp.uint16), jnp.bfloat16
    )
    o_ref[...] = b16.astype(jnp.float32) + bias_ref[...]


@jax.jit
def _tc_select(g, s2d, b2d):
    block = 2048
    grid = (BATCH // block,)
    return pl.pallas_call(
        _select_body,
        grid=grid,
        in_specs=[
            pl.BlockSpec((block, 2 * EMB_DIM), lambda i: (i, 0)),
            pl.BlockSpec((block, 1), lambda i: (i, 0)),
            pl.BlockSpec((1, NUM_CLASS), lambda i: (0, 0)),
        ],
        out_specs=pl.BlockSpec((block, NUM_CLASS), lambda i: (i, 0)),
        out_shape=jax.ShapeDtypeStruct((BATCH, NUM_CLASS), jnp.float32),
    )(g, s2d, b2d)


def kernel(node, emb, fc_w, fc_b):
    node = node.astype(jnp.int32)
    s = node // Q
    t = node - s * Q
    tidx3d = t.reshape(NUM_WORKERS, N_CHUNKS, CHUNK)
    s2d = s.reshape(BATCH, 1)
    y = _tc_project(emb.T, fc_w)
    g = _sc_gather(y, tidx3d).reshape(BATCH, 2 * EMB_DIM)
    return _tc_select(g, s2d, fc_b.reshape(1, NUM_CLASS))
